# baseline (speedup 1.0000x reference)
import functools

import jax
import jax.numpy as jnp
from jax import lax
from jax.experimental import pallas as pl
from jax.experimental.pallas import tpu as pltpu
from jax.experimental.pallas import tpu_sc as plsc

B = 16384
D = 32
NC, NS = 2, 16
NW = NC * NS
PER_W = B // NW
PACK = 4
ROWS_W = PER_W // PACK

_mesh = plsc.VectorSubcoreMesh(core_axis_name="c", subcore_axis_name="s")


@functools.partial(
    pl.kernel,
    out_type=[
        jax.ShapeDtypeStruct((2, B // PACK, 128), jnp.float32),
        jax.ShapeDtypeStruct((4, B // PACK, 128), jnp.float32),
    ],
    mesh=_mesh,
    scratch_types=[
        pltpu.VMEM((PER_W,), jnp.int32),
        pltpu.VMEM((PER_W, 128), jnp.float32),
        pltpu.VMEM((ROWS_W, 128), jnp.float32),
        pltpu.SemaphoreType.DMA,
    ],
    compiler_params=pltpu.CompilerParams(use_tc_tiling_on_sc=False),
)
def _sc_gather(t_tab, a_tab, t_idx, a_idx, t_out, a_out,
               idx_v, big_v, sel_v, sem):
    wid = lax.axis_index("s") * NC + lax.axis_index("c")
    base = wid * PER_W
    obase = wid * ROWS_W

    def one_slot(tab, idx_hbm, out_hbm, j):
        pltpu.sync_copy(idx_hbm.at[j].at[pl.ds(base, PER_W)], idx_v)
        pltpu.async_copy(tab.at[idx_v], big_v, sem).wait()

        @pl.loop(0, ROWS_W)
        def _(r):
            for k in range(4):
                b = 4 * r + k
                sel_v[r, pl.ds(k * 32, 16)] = big_v[b, pl.ds(0, 16)]
                sel_v[r, pl.ds(k * 32 + 16, 16)] = big_v[b, pl.ds(16, 16)]

        pltpu.sync_copy(sel_v, out_hbm.at[j].at[pl.ds(obase, ROWS_W)])

    for j in range(2):
        one_slot(t_tab, t_idx, t_out, j)
    for j in range(4):
        one_slot(a_tab, a_idx, a_out, j)


NP = 4000


def _pad_body(x_ref, o_ref):
    o_ref[:, 0:D] = x_ref[...]


def _widen(tab, n_rows):
    np_rows = NP if n_rows >= NP else n_rows
    return pl.pallas_call(
        _pad_body,
        grid=(n_rows // np_rows,),
        in_specs=[pl.BlockSpec((np_rows, D), lambda i: (i, 0))],
        out_specs=pl.BlockSpec((np_rows, 128), lambda i: (i, 0)),
        out_shape=jax.ShapeDtypeStruct((n_rows, 128), jnp.float32),
    )(tab)


NBP = 1024


def _combine_body(t_ref, a_ref, wd_ref, b_ref, o_ref):
    acc = jnp.dot(t_ref[0], wd_ref[0], preferred_element_type=jnp.float32)
    acc = acc + jnp.dot(t_ref[1], wd_ref[1], preferred_element_type=jnp.float32)
    for j in range(4):
        acc = acc + jnp.dot(a_ref[j], wd_ref[2 + j],
                            preferred_element_type=jnp.float32)
    o_ref[...] = acc + b_ref[...]


def _combine(t_emb, a_emb, wd, bp):
    return pl.pallas_call(
        _combine_body,
        grid=(B // PACK // NBP,),
        in_specs=[
            pl.BlockSpec((2, NBP, 128), lambda i: (0, i, 0)),
            pl.BlockSpec((4, NBP, 128), lambda i: (0, i, 0)),
            pl.BlockSpec((6, 128, 128), lambda i: (0, 0, 0)),
            pl.BlockSpec((1, 128), lambda i: (0, 0)),
        ],
        out_specs=pl.BlockSpec((NBP, 128), lambda i: (i, 0)),
        out_shape=jax.ShapeDtypeStruct((B // PACK, 128), jnp.float32),
    )(t_emb, a_emb, wd, bp)


def kernel(type_ids, ability_ids, type_table, ability_table, W, b):
    t_idx = type_ids.T.astype(jnp.int32)
    a_idx = ability_ids.T.astype(jnp.int32)
    t_tab = _widen(type_table, 1000)
    a_tab = _widen(ability_table, 1000000)
    t_emb, a_emb = _sc_gather(t_tab, a_tab, t_idx, a_idx)

    wt = W.T
    eye4 = jnp.eye(PACK, dtype=W.dtype)
    wd = jnp.stack([jnp.kron(eye4, wt[j * D:(j + 1) * D, :])
                    for j in range(6)])
    bp = jnp.tile(b, PACK).reshape(1, 128)
    out = _combine(t_emb, a_emb, wd, bp)
    return out.reshape(B, D)


# speedup vs baseline: 1.3221x; 1.3221x over previous
import functools

import jax
import jax.numpy as jnp
from jax import lax
from jax.experimental import pallas as pl
from jax.experimental.pallas import tpu as pltpu
from jax.experimental.pallas import tpu_sc as plsc

B = 16384
D = 32
NA = 1000000
NT = 1000
NC, NS = 2, 16
NW = NC * NS
PER_W = B // NW
PACK = 4
ROWS_W = PER_W // PACK

_mesh = plsc.VectorSubcoreMesh(core_axis_name="c", subcore_axis_name="s")


NPRJ = 2048


def _project_body(tT_ref, v_ref, o_ref):
    o_ref[...] = jax.lax.dot_general(
        tT_ref[...], v_ref[...], (((0,), (0,)), ((), ())),
        preferred_element_type=jnp.float32)


def _project(tT, vcat, n_rows):
    blk = NPRJ if n_rows >= NPRJ else n_rows
    return pl.pallas_call(
        _project_body,
        grid=(pl.cdiv(n_rows, blk),),
        in_specs=[
            pl.BlockSpec((D, blk), lambda i: (0, i)),
            pl.BlockSpec((D, 128), lambda i: (0, 0)),
        ],
        out_specs=pl.BlockSpec((blk, 128), lambda i: (i, 0)),
        out_shape=jax.ShapeDtypeStruct((n_rows, 128), jnp.float32),
    )(tT, vcat)


@functools.partial(
    pl.kernel,
    out_type=jax.ShapeDtypeStruct((B // PACK, 128), jnp.float32),
    mesh=_mesh,
    scratch_types=[
        pltpu.VMEM((PER_W,), jnp.int32),
        pltpu.VMEM((PER_W, 128), jnp.float32),
        pltpu.VMEM((ROWS_W, 128), jnp.float32),
        pltpu.SemaphoreType.DMA,
    ],
    compiler_params=pltpu.CompilerParams(use_tc_tiling_on_sc=False),
)
def _sc_gather_sum(tp_tab, p_tab, t_idx, a_idx, out,
                   idx_v, big_v, acc_v, sem):
    wid = lax.axis_index("s") * NC + lax.axis_index("c")
    base = wid * PER_W
    obase = wid * ROWS_W

    def one_slot(tab, idx_hbm, j, lane0, first):
        pltpu.sync_copy(idx_hbm.at[j].at[pl.ds(base, PER_W)], idx_v)
        pltpu.async_copy(tab.at[idx_v], big_v, sem).wait()

        @pl.loop(0, PER_W)
        def _(bb):
            r = bb >> 2
            k = bb & 3
            lo = big_v[bb, pl.ds(lane0, 16)]
            hi = big_v[bb, pl.ds(lane0 + 16, 16)]
            if first:
                acc_v[r, pl.ds(k * 32, 16)] = lo
                acc_v[r, pl.ds(k * 32 + 16, 16)] = hi
            else:
                acc_v[r, pl.ds(k * 32, 16)] += lo
                acc_v[r, pl.ds(k * 32 + 16, 16)] += hi

    one_slot(tp_tab, t_idx, 0, 0, True)
    one_slot(tp_tab, t_idx, 1, 32, False)
    for j in range(4):
        one_slot(p_tab, a_idx, j, j * 32, False)

    pltpu.sync_copy(acc_v, out.at[pl.ds(obase, ROWS_W)])


def kernel(type_ids, ability_ids, type_table, ability_table, W, b):
    t_idx = type_ids.T.astype(jnp.int32)
    a_idx = ability_ids.T.astype(jnp.int32)

    wt = W.T
    vcat_a = jnp.concatenate([wt[(2 + j) * D:(3 + j) * D, :]
                              for j in range(4)], axis=1)
    vcat_t = jnp.concatenate(
        [wt[0:D, :], wt[D:2 * D, :], jnp.zeros((D, 64), W.dtype)], axis=1)

    p_tab = _project(ability_table.T, vcat_a, NA)
    tp_tab = _project(type_table.T, vcat_t, NT)
    bias_row = jnp.concatenate([b, jnp.zeros((96,), b.dtype)]).reshape(1, 128)
    tp_tab = tp_tab + bias_row

    out = _sc_gather_sum(tp_tab, p_tab, t_idx, a_idx)
    return out.reshape(B, D)


# speedup vs baseline: 1.8540x; 1.4023x over previous
import functools

import jax
import jax.numpy as jnp
from jax import lax
from jax.experimental import pallas as pl
from jax.experimental.pallas import tpu as pltpu
from jax.experimental.pallas import tpu_sc as plsc

B = 16384
D = 32
NA = 1000000
NT = 1000
NC, NS = 2, 16
NW = NC * NS
PER_W = B // NW
PACK = 4
ROWS_W = PER_W // PACK

_mesh = plsc.VectorSubcoreMesh(core_axis_name="c", subcore_axis_name="s")


NPRJ = 4096


def _project_body(tT_ref, v_ref, o_ref):
    o_ref[...] = jax.lax.dot_general(
        tT_ref[...].astype(jnp.bfloat16), v_ref[...].astype(jnp.bfloat16),
        (((0,), (0,)), ((), ())),
        preferred_element_type=jnp.float32)


def _project(tT, vcat, n_rows):
    blk = NPRJ if n_rows >= NPRJ else n_rows
    return pl.pallas_call(
        _project_body,
        grid=(pl.cdiv(n_rows, blk),),
        in_specs=[
            pl.BlockSpec((D, blk), lambda i: (0, i)),
            pl.BlockSpec((D, 128), lambda i: (0, 0)),
        ],
        out_specs=pl.BlockSpec((blk, 128), lambda i: (i, 0)),
        out_shape=jax.ShapeDtypeStruct((n_rows, 128), jnp.float32),
    )(tT, vcat)


@functools.partial(
    pl.kernel,
    out_type=jax.ShapeDtypeStruct((B // PACK, 128), jnp.float32),
    mesh=_mesh,
    scratch_types=[
        pltpu.VMEM((PER_W,), jnp.int32),
        pltpu.VMEM((PER_W, 128), jnp.float32),
        pltpu.VMEM((ROWS_W, 128), jnp.float32),
        pltpu.SemaphoreType.DMA,
    ],
    compiler_params=pltpu.CompilerParams(use_tc_tiling_on_sc=False),
)
def _sc_gather_sum(tp_tab, p_tab, t_idx, a_idx, out,
                   idx_v, big_v, acc_v, sem):
    wid = lax.axis_index("s") * NC + lax.axis_index("c")
    base = wid * PER_W
    obase = wid * ROWS_W

    def one_slot(tab, idx_hbm, j, lane0, first):
        pltpu.sync_copy(idx_hbm.at[j].at[pl.ds(base, PER_W)], idx_v)
        pltpu.async_copy(tab.at[idx_v], big_v, sem).wait()

        @pl.loop(0, PER_W)
        def _(bb):
            r = bb >> 2
            k = bb & 3
            lo = big_v[bb, pl.ds(lane0, 16)]
            hi = big_v[bb, pl.ds(lane0 + 16, 16)]
            if first:
                acc_v[r, pl.ds(k * 32, 16)] = lo
                acc_v[r, pl.ds(k * 32 + 16, 16)] = hi
            else:
                acc_v[r, pl.ds(k * 32, 16)] += lo
                acc_v[r, pl.ds(k * 32 + 16, 16)] += hi

    one_slot(tp_tab, t_idx, 0, 0, True)
    one_slot(tp_tab, t_idx, 1, 32, False)
    for j in range(4):
        one_slot(p_tab, a_idx, j, j * 32, False)

    pltpu.sync_copy(acc_v, out.at[pl.ds(obase, ROWS_W)])


def kernel(type_ids, ability_ids, type_table, ability_table, W, b):
    t_idx = type_ids.T.astype(jnp.int32)
    a_idx = ability_ids.T.astype(jnp.int32)

    wt = W.T
    vcat_a = jnp.concatenate([wt[(2 + j) * D:(3 + j) * D, :]
                              for j in range(4)], axis=1)
    vcat_t = jnp.concatenate(
        [wt[0:D, :], wt[D:2 * D, :], jnp.zeros((D, 64), W.dtype)], axis=1)

    p_tab = _project(ability_table.T, vcat_a, NA)
    tp_tab = _project(type_table.T, vcat_t, NT)
    bias_row = jnp.concatenate([b, jnp.zeros((96,), b.dtype)]).reshape(1, 128)
    tp_tab = tp_tab + bias_row

    out = _sc_gather_sum(tp_tab, p_tab, t_idx, a_idx)
    return out.reshape(B, D)


# speedup vs baseline: 2.3160x; 1.2492x over previous
import functools

import jax
import jax.numpy as jnp
from jax import lax
from jax.experimental import pallas as pl
from jax.experimental.pallas import tpu as pltpu
from jax.experimental.pallas import tpu_sc as plsc

B = 16384
D = 32
NA = 1000000
NT = 1000
NC, NS = 2, 16
NW = NC * NS
PER_W = B // NW
PACK = 4
ROWS_W = PER_W // PACK

_mesh = plsc.VectorSubcoreMesh(core_axis_name="c", subcore_axis_name="s")


NPRJ = 8192


def _project_body(tT_ref, v_ref, o_ref):
    o_ref[...] = jax.lax.dot_general(
        tT_ref[...].astype(jnp.bfloat16), v_ref[...].astype(jnp.bfloat16),
        (((0,), (0,)), ((), ())),
        preferred_element_type=jnp.float32)


def _project(tT, vcat, n_rows):
    blk = NPRJ if n_rows >= NPRJ else n_rows
    return pl.pallas_call(
        _project_body,
        grid=(pl.cdiv(n_rows, blk),),
        in_specs=[
            pl.BlockSpec((D, blk), lambda i: (0, i)),
            pl.BlockSpec((D, 128), lambda i: (0, 0)),
        ],
        out_specs=pl.BlockSpec((blk, 128), lambda i: (i, 0)),
        out_shape=jax.ShapeDtypeStruct((n_rows, 128), jnp.float32),
    )(tT, vcat)


def _gather_sum(tab, idx, n_slots, lane0s):
    @functools.partial(
        pl.kernel,
        out_type=jax.ShapeDtypeStruct((B // PACK, 128), jnp.float32),
        mesh=_mesh,
        scratch_types=[
            pltpu.VMEM((PER_W,), jnp.int32),
            pltpu.VMEM((PER_W, 128), jnp.float32),
            pltpu.VMEM((ROWS_W, 128), jnp.float32),
            pltpu.SemaphoreType.DMA,
        ],
        compiler_params=pltpu.CompilerParams(use_tc_tiling_on_sc=False),
    )
    def k(tab_hbm, idx_hbm, out, idx_v, big_v, acc_v, sem):
        wid = lax.axis_index("s") * NC + lax.axis_index("c")
        base = wid * PER_W
        obase = wid * ROWS_W

        for j in range(n_slots):
            pltpu.sync_copy(idx_hbm.at[j].at[pl.ds(base, PER_W)], idx_v)
            pltpu.async_copy(tab_hbm.at[idx_v], big_v, sem).wait()
            lane0 = lane0s[j]
            first = j == 0

            @pl.loop(0, PER_W)
            def _(bb):
                r = bb >> 2
                k_ = bb & 3
                lo = big_v[bb, pl.ds(lane0, 16)]
                hi = big_v[bb, pl.ds(lane0 + 16, 16)]
                if first:
                    acc_v[r, pl.ds(k_ * 32, 16)] = lo
                    acc_v[r, pl.ds(k_ * 32 + 16, 16)] = hi
                else:
                    acc_v[r, pl.ds(k_ * 32, 16)] += lo
                    acc_v[r, pl.ds(k_ * 32 + 16, 16)] += hi

        pltpu.sync_copy(acc_v, out.at[pl.ds(obase, ROWS_W)])

    return k(tab, idx)


def kernel(type_ids, ability_ids, type_table, ability_table, W, b):
    t_idx = type_ids.T.astype(jnp.int32)
    a_idx = ability_ids.T.astype(jnp.int32)

    wt = W.T
    vcat_a = jnp.concatenate([wt[(2 + j) * D:(3 + j) * D, :]
                              for j in range(4)], axis=1)
    vcat_t = jnp.concatenate(
        [wt[0:D, :], wt[D:2 * D, :], jnp.zeros((D, 64), W.dtype)], axis=1)

    p_tab = _project(ability_table.T, vcat_a, NA)
    tp_tab = _project(type_table.T, vcat_t, NT)
    bias_row = jnp.concatenate([b, jnp.zeros((96,), b.dtype)]).reshape(1, 128)
    tp_tab = tp_tab + bias_row

    out_t = _gather_sum(tp_tab, t_idx, 2, (0, 32))
    out_a = _gather_sum(p_tab, a_idx, 4, (0, 32, 64, 96))
    return (out_t + out_a).reshape(B, D)


# speedup vs baseline: 2.4597x; 1.0620x over previous
import functools

import jax
import jax.numpy as jnp
from jax import lax
from jax.experimental import pallas as pl
from jax.experimental.pallas import tpu as pltpu
from jax.experimental.pallas import tpu_sc as plsc

B = 16384
D = 32
NA = 1000000
NT = 1000
NC, NS = 2, 16
NW = NC * NS
PER_W = B // NW
PACK = 4
ROWS_W = PER_W // PACK

_mesh = plsc.VectorSubcoreMesh(core_axis_name="c", subcore_axis_name="s")


NPRJ = 16384


def _project_body(tT_ref, v_ref, o_ref):
    o_ref[...] = jax.lax.dot_general(
        tT_ref[...].astype(jnp.bfloat16), v_ref[...].astype(jnp.bfloat16),
        (((0,), (0,)), ((), ())),
        preferred_element_type=jnp.float32)


def _project(tT, vcat, n_rows):
    blk = NPRJ if n_rows >= NPRJ else n_rows
    return pl.pallas_call(
        _project_body,
        grid=(pl.cdiv(n_rows, blk),),
        in_specs=[
            pl.BlockSpec((D, blk), lambda i: (0, i)),
            pl.BlockSpec((D, 128), lambda i: (0, 0)),
        ],
        out_specs=pl.BlockSpec((blk, 128), lambda i: (i, 0)),
        out_shape=jax.ShapeDtypeStruct((n_rows, 128), jnp.float32),
    )(tT, vcat)


def _gather_sum(tab, idx, n_slots, lane0s):
    @functools.partial(
        pl.kernel,
        out_type=jax.ShapeDtypeStruct((B // PACK, 128), jnp.float32),
        mesh=_mesh,
        scratch_types=[
            pltpu.VMEM((PER_W,), jnp.int32),
            pltpu.VMEM((PER_W, 128), jnp.float32),
            pltpu.VMEM((ROWS_W, 128), jnp.float32),
            pltpu.SemaphoreType.DMA,
        ],
        compiler_params=pltpu.CompilerParams(use_tc_tiling_on_sc=False),
    )
    def k(tab_hbm, idx_hbm, out, idx_v, big_v, acc_v, sem):
        wid = lax.axis_index("s") * NC + lax.axis_index("c")
        base = wid * PER_W
        obase = wid * ROWS_W

        for j in range(n_slots):
            pltpu.sync_copy(idx_hbm.at[j].at[pl.ds(base, PER_W)], idx_v)
            pltpu.async_copy(tab_hbm.at[idx_v], big_v, sem).wait()
            lane0 = lane0s[j]
            first = j == 0

            @pl.loop(0, PER_W)
            def _(bb):
                r = bb >> 2
                k_ = bb & 3
                lo = big_v[bb, pl.ds(lane0, 16)]
                hi = big_v[bb, pl.ds(lane0 + 16, 16)]
                if first:
                    acc_v[r, pl.ds(k_ * 32, 16)] = lo
                    acc_v[r, pl.ds(k_ * 32 + 16, 16)] = hi
                else:
                    acc_v[r, pl.ds(k_ * 32, 16)] += lo
                    acc_v[r, pl.ds(k_ * 32 + 16, 16)] += hi

        pltpu.sync_copy(acc_v, out.at[pl.ds(obase, ROWS_W)])

    return k(tab, idx)


def kernel(type_ids, ability_ids, type_table, ability_table, W, b):
    t_idx = type_ids.T.astype(jnp.int32)
    a_idx = ability_ids.T.astype(jnp.int32)

    wt = W.T
    vcat_a = jnp.concatenate([wt[(2 + j) * D:(3 + j) * D, :]
                              for j in range(4)], axis=1)
    vcat_t = jnp.concatenate(
        [wt[0:D, :], wt[D:2 * D, :], jnp.zeros((D, 64), W.dtype)], axis=1)

    tp_tab = _project(type_table.T, vcat_t, NT)
    bias_row = jnp.concatenate([b, jnp.zeros((96,), b.dtype)]).reshape(1, 128)
    tp_tab = tp_tab + bias_row
    p_tab = _project(ability_table.T, vcat_a, NA)

    out_t = _gather_sum(tp_tab, t_idx, 2, (0, 32))
    out_a = _gather_sum(p_tab, a_idx, 4, (0, 32, 64, 96))
    return (out_t + out_a).reshape(B, D)
